# baseline (speedup 1.0000x reference)
import functools

import jax
import jax.numpy as jnp
from jax.experimental import pallas as pl
import jax.experimental.pallas.tpu as pltpu

GROUPS_K = 4
BETA_K = 0.25


def _vq_body(z_ref, cb_ref, quant_ref, inds_ref, counts_ref, lsum_ref,
             lacc_ref, *, n_b, n_s, n_e):
    g = pl.program_id(0)
    b = pl.program_id(1)
    s = pl.program_id(2)
    first_of_group = jnp.logical_and(b == 0, s == 0)
    last = jnp.logical_and(g == GROUPS_K - 1,
                           jnp.logical_and(b == n_b - 1, s == n_s - 1))

    @pl.when(jnp.logical_and(g == 0, first_of_group))
    def _init_all():
        lacc_ref[0] = 0.0

    xT = z_ref[0]
    cb = cb_ref[0]
    cxx = jnp.sum(cb * cb, axis=1, keepdims=True)
    cb2 = cb * (-2.0)
    scoresT = jax.lax.dot_general(
        cb2, xT, (((1,), (0,)), ((), ())),
        preferred_element_type=jnp.float32)
    d2 = scoresT + cxx
    minv = jnp.min(d2, axis=0, keepdims=True)
    iota = jax.lax.broadcasted_iota(jnp.int32, d2.shape, 0)
    idx = jnp.min(jnp.where(d2 == minv, iota, n_e), axis=0)
    inds_ref[0, 0, 0, :] = idx
    oh = (iota == idx[None, :]).astype(jnp.bfloat16)
    ones_row = jnp.ones((1, xT.shape[1]), jnp.bfloat16)
    cnt = jax.lax.dot_general(
        ones_row, oh, (((1,), (1,)), ((), ())),
        preferred_element_type=jnp.float32)
    counts_ref[0] = jnp.where(first_of_group, cnt, counts_ref[0] + cnt)
    cb_hi = cb.astype(jnp.bfloat16)
    cb_lo = (cb - cb_hi.astype(jnp.float32)).astype(jnp.bfloat16)
    zqT = (jax.lax.dot_general(
        cb_hi, oh, (((0,), (0,)), ((), ())),
        preferred_element_type=jnp.float32)
        + jax.lax.dot_general(
        cb_lo, oh, (((0,), (0,)), ((), ())),
        preferred_element_type=jnp.float32))
    quant_ref[0] = zqT
    lacc_ref[0] += jnp.sum((zqT - xT) ** 2)

    @pl.when(last)
    def _fin():
        lsum_ref[:, :] = jnp.full((1, 1), lacc_ref[0], jnp.float32)


def _fin_body(counts_ref, lsum_ref, loss_ref, perps_ref, *, t, n_elems):
    counts = counts_ref[:, 0, :]
    probs = counts * (1.0 / t)
    ent = -jnp.sum(probs * jnp.log(probs + 1e-10), axis=1, keepdims=True)
    perps_ref[:, :] = jnp.exp(ent)
    loss_ref[:, :] = (1.0 + BETA_K) / n_elems * lsum_ref[:, :]


def kernel(z, codebooks):
    z = z.astype(jnp.float32)
    B, C, H, W = z.shape
    HW = H * W
    G, N_E, DPG = codebooks.shape
    zr = z.reshape(B, C, HW)
    S = min(2048, HW)
    n_s = HW // S
    grid = (G, B, n_s)
    T = B * HW
    n_elems = T * DPG

    body = functools.partial(_vq_body, n_b=B, n_s=n_s, n_e=N_E)
    quant, inds4, counts, lsum = pl.pallas_call(
        body,
        grid=grid,
        in_specs=[
            pl.BlockSpec((1, DPG, S), lambda g, b, s: (b, g, s)),
            pl.BlockSpec((1, N_E, DPG), lambda g, b, s: (g, 0, 0)),
        ],
        out_specs=[
            pl.BlockSpec((1, DPG, S), lambda g, b, s: (b, g, s)),
            pl.BlockSpec((1, 1, 1, S), lambda g, b, s: (g, b, 0, s)),
            pl.BlockSpec((1, 1, N_E), lambda g, b, s: (g, 0, 0)),
            pl.BlockSpec((1, 1), lambda g, b, s: (0, 0)),
        ],
        out_shape=[
            jax.ShapeDtypeStruct((B, C, HW), jnp.float32),
            jax.ShapeDtypeStruct((G, B, 1, HW), jnp.int32),
            jax.ShapeDtypeStruct((G, 1, N_E), jnp.float32),
            jax.ShapeDtypeStruct((1, 1), jnp.float32),
        ],
        scratch_shapes=[
            pltpu.SMEM((1,), jnp.float32),
        ],
    )(zr, codebooks)

    fin = functools.partial(_fin_body, t=float(T), n_elems=float(n_elems))
    loss, perps = pl.pallas_call(
        fin,
        out_shape=[
            jax.ShapeDtypeStruct((1, 1), jnp.float32),
            jax.ShapeDtypeStruct((G, 1), jnp.float32),
        ],
    )(counts, lsum)

    quantized = quant.reshape(B, C, H, W)
    inds = inds4.reshape(G, B, HW)
    return (quantized, loss[0, 0], perps[:, 0], inds)


# speedup vs baseline: 1.0413x; 1.0413x over previous
import functools

import jax
import jax.numpy as jnp
from jax.experimental import pallas as pl
import jax.experimental.pallas.tpu as pltpu

GROUPS_K = 4
BETA_K = 0.25


def _vq_body(z_ref, cb_ref, quant_ref, inds_ref, loss_ref, perps_ref,
             counts_ref, lacc_ref, *, n_b, n_s, n_e, n_elems):
    g = pl.program_id(0)
    b = pl.program_id(1)
    s = pl.program_id(2)
    first_of_group = jnp.logical_and(b == 0, s == 0)
    last_of_group = jnp.logical_and(b == n_b - 1, s == n_s - 1)

    @pl.when(jnp.logical_and(g == 0, first_of_group))
    def _init_all():
        lacc_ref[0] = 0.0

    xT = z_ref[0]
    cb = cb_ref[0]
    cxx = jnp.sum(cb * cb, axis=1, keepdims=True)
    cb2 = cb * (-2.0)
    scoresT = jax.lax.dot_general(
        cb2, xT, (((1,), (0,)), ((), ())),
        preferred_element_type=jnp.float32)
    d2 = scoresT + cxx
    minv = jnp.min(d2, axis=0, keepdims=True)
    iota = jax.lax.broadcasted_iota(jnp.int32, d2.shape, 0)
    idx = jnp.min(jnp.where(d2 == minv, iota, n_e), axis=0)
    inds_ref[0, 0, 0, :] = idx
    oh = (iota == idx[None, :]).astype(jnp.bfloat16)
    ones_row = jnp.ones((1, xT.shape[1]), jnp.bfloat16)
    cnt = jax.lax.dot_general(
        ones_row, oh, (((1,), (1,)), ((), ())),
        preferred_element_type=jnp.float32)
    counts_ref[:, :] = jnp.where(first_of_group, cnt, counts_ref[:, :] + cnt)
    cb_hi = cb.astype(jnp.bfloat16)
    cb_lo = (cb - cb_hi.astype(jnp.float32)).astype(jnp.bfloat16)
    zqT = (jax.lax.dot_general(
        cb_hi, oh, (((0,), (0,)), ((), ())),
        preferred_element_type=jnp.float32)
        + jax.lax.dot_general(
        cb_lo, oh, (((0,), (0,)), ((), ())),
        preferred_element_type=jnp.float32))
    quant_ref[0] = zqT
    lacc_ref[0] += jnp.sum((zqT - xT) ** 2)

    @pl.when(last_of_group)
    def _fin_group():
        t = n_b * n_s * xT.shape[1]
        probs = counts_ref[0, :] * (1.0 / t)
        ent = -jnp.sum(probs * jnp.log(probs + 1e-10))
        lane4 = jax.lax.broadcasted_iota(jnp.int32, (GROUPS_K,), 0)
        prev = jnp.where(g == 0, jnp.zeros((GROUPS_K,), jnp.float32),
                         perps_ref[0, :])
        perps_ref[0, :] = jnp.where(lane4 == g, jnp.exp(ent), prev)

    @pl.when(jnp.logical_and(g == GROUPS_K - 1, last_of_group))
    def _fin_all():
        total = (1.0 + BETA_K) * lacc_ref[0] / n_elems
        loss_ref[:, :] = jnp.full((1, 1), total, jnp.float32)


def kernel(z, codebooks):
    z = z.astype(jnp.float32)
    B, C, H, W = z.shape
    HW = H * W
    G, N_E, DPG = codebooks.shape
    zr = z.reshape(B, C, HW)
    S = min(4096, HW)
    n_s = HW // S
    grid = (G, B, n_s)
    T = B * HW
    n_elems = T * DPG

    body = functools.partial(_vq_body, n_b=B, n_s=n_s, n_e=N_E,
                             n_elems=float(n_elems))
    quant, inds4, loss, perps = pl.pallas_call(
        body,
        grid=grid,
        in_specs=[
            pl.BlockSpec((1, DPG, S), lambda g, b, s: (b, g, s)),
            pl.BlockSpec((1, N_E, DPG), lambda g, b, s: (g, 0, 0)),
        ],
        out_specs=[
            pl.BlockSpec((1, DPG, S), lambda g, b, s: (b, g, s)),
            pl.BlockSpec((1, 1, 1, S), lambda g, b, s: (g, b, 0, s)),
            pl.BlockSpec((1, 1), lambda g, b, s: (0, 0)),
            pl.BlockSpec((1, GROUPS_K), lambda g, b, s: (0, 0)),
        ],
        out_shape=[
            jax.ShapeDtypeStruct((B, C, HW), jnp.float32),
            jax.ShapeDtypeStruct((G, B, 1, HW), jnp.int32),
            jax.ShapeDtypeStruct((1, 1), jnp.float32),
            jax.ShapeDtypeStruct((1, GROUPS_K), jnp.float32),
        ],
        scratch_shapes=[
            pltpu.VMEM((1, N_E), jnp.float32),
            pltpu.SMEM((1,), jnp.float32),
        ],
    )(zr, codebooks)

    quantized = quant.reshape(B, C, H, W)
    inds = inds4.reshape(G, B, HW)
    return (quantized, loss[0, 0], perps[0], inds)


# speedup vs baseline: 1.2231x; 1.1745x over previous
import functools

import jax
import jax.numpy as jnp
from jax.experimental import pallas as pl
import jax.experimental.pallas.tpu as pltpu

GROUPS_K = 4
BETA_K = 0.25


def _vq_body(z_ref, cb_ref, quant_ref, inds_ref, loss_ref, perps_ref,
             counts_ref, lacc_ref, *, n_b, n_s, n_e, n_elems):
    g = pl.program_id(0)
    b = pl.program_id(1)
    s = pl.program_id(2)
    first_of_group = jnp.logical_and(b == 0, s == 0)
    last_of_group = jnp.logical_and(b == n_b - 1, s == n_s - 1)

    @pl.when(jnp.logical_and(g == 0, first_of_group))
    def _init_all():
        lacc_ref[0] = 0.0

    xT = z_ref[0]
    cb = cb_ref[0]
    cxx = jnp.sum(cb * cb, axis=1, keepdims=True)
    cb2 = cb * (-2.0)
    scoresT = jax.lax.dot_general(
        cb2, xT, (((1,), (0,)), ((), ())),
        preferred_element_type=jnp.float32)
    d2 = scoresT + cxx
    iota = jax.lax.broadcasted_iota(jnp.int32, d2.shape, 0)
    idx = jnp.argmin(d2, axis=0).astype(jnp.int32)
    inds_ref[0, 0, 0, :] = idx
    oh = (iota == idx[None, :]).astype(jnp.bfloat16)
    ones_row = jnp.ones((1, xT.shape[1]), jnp.bfloat16)
    cnt = jax.lax.dot_general(
        ones_row, oh, (((1,), (1,)), ((), ())),
        preferred_element_type=jnp.float32)
    counts_ref[:, :] = jnp.where(first_of_group, cnt, counts_ref[:, :] + cnt)
    cb_hi = cb.astype(jnp.bfloat16)
    cb_lo = (cb - cb_hi.astype(jnp.float32)).astype(jnp.bfloat16)
    zqT = (jax.lax.dot_general(
        cb_hi, oh, (((0,), (0,)), ((), ())),
        preferred_element_type=jnp.float32)
        + jax.lax.dot_general(
        cb_lo, oh, (((0,), (0,)), ((), ())),
        preferred_element_type=jnp.float32))
    quant_ref[0] = zqT
    lacc_ref[0] += jnp.sum((zqT - xT) ** 2)

    @pl.when(last_of_group)
    def _fin_group():
        t = n_b * n_s * xT.shape[1]
        probs = counts_ref[0, :] * (1.0 / t)
        ent = -jnp.sum(probs * jnp.log(probs + 1e-10))
        lane4 = jax.lax.broadcasted_iota(jnp.int32, (GROUPS_K,), 0)
        prev = jnp.where(g == 0, jnp.zeros((GROUPS_K,), jnp.float32),
                         perps_ref[0, :])
        perps_ref[0, :] = jnp.where(lane4 == g, jnp.exp(ent), prev)

    @pl.when(jnp.logical_and(g == GROUPS_K - 1, last_of_group))
    def _fin_all():
        total = (1.0 + BETA_K) * lacc_ref[0] / n_elems
        loss_ref[:, :] = jnp.full((1, 1), total, jnp.float32)


def kernel(z, codebooks):
    z = z.astype(jnp.float32)
    B, C, H, W = z.shape
    HW = H * W
    G, N_E, DPG = codebooks.shape
    zr = z.reshape(B, C, HW)
    S = min(4096, HW)
    n_s = HW // S
    grid = (G, B, n_s)
    T = B * HW
    n_elems = T * DPG

    body = functools.partial(_vq_body, n_b=B, n_s=n_s, n_e=N_E,
                             n_elems=float(n_elems))
    quant, inds4, loss, perps = pl.pallas_call(
        body,
        grid=grid,
        in_specs=[
            pl.BlockSpec((1, DPG, S), lambda g, b, s: (b, g, s)),
            pl.BlockSpec((1, N_E, DPG), lambda g, b, s: (g, 0, 0)),
        ],
        out_specs=[
            pl.BlockSpec((1, DPG, S), lambda g, b, s: (b, g, s)),
            pl.BlockSpec((1, 1, 1, S), lambda g, b, s: (g, b, 0, s)),
            pl.BlockSpec((1, 1), lambda g, b, s: (0, 0)),
            pl.BlockSpec((1, GROUPS_K), lambda g, b, s: (0, 0)),
        ],
        out_shape=[
            jax.ShapeDtypeStruct((B, C, HW), jnp.float32),
            jax.ShapeDtypeStruct((G, B, 1, HW), jnp.int32),
            jax.ShapeDtypeStruct((1, 1), jnp.float32),
            jax.ShapeDtypeStruct((1, GROUPS_K), jnp.float32),
        ],
        scratch_shapes=[
            pltpu.VMEM((1, N_E), jnp.float32),
            pltpu.SMEM((1,), jnp.float32),
        ],
    )(zr, codebooks)

    quantized = quant.reshape(B, C, H, W)
    inds = inds4.reshape(G, B, HW)
    return (quantized, loss[0, 0], perps[0], inds)


# speedup vs baseline: 1.4474x; 1.1834x over previous
import functools

import jax
import jax.numpy as jnp
from jax.experimental import pallas as pl
import jax.experimental.pallas.tpu as pltpu

GROUPS_K = 4
BETA_K = 0.25


def _vq_body(z_ref, cb_ref, quant_ref, inds_ref, loss_ref, perps_ref,
             counts_ref, lacc_ref, *, n_b, n_s, n_e, n_elems):
    g = pl.program_id(0)
    b = pl.program_id(1)
    s = pl.program_id(2)
    first_of_group = jnp.logical_and(b == 0, s == 0)
    last_of_group = jnp.logical_and(b == n_b - 1, s == n_s - 1)

    @pl.when(jnp.logical_and(g == 0, first_of_group))
    def _init_all():
        lacc_ref[0] = 0.0

    xT = z_ref[0]
    cb = cb_ref[0]
    cxx = jnp.sum(cb * cb, axis=1, keepdims=True)
    cb2 = cb * (-2.0)
    scoresT = jax.lax.dot_general(
        cb2, xT, (((1,), (0,)), ((), ())),
        preferred_element_type=jnp.float32)
    d2 = scoresT + cxx
    iota = jax.lax.broadcasted_iota(jnp.int32, d2.shape, 0)
    idx = jnp.argmin(d2, axis=0).astype(jnp.int32)
    inds_ref[0, 0, 0, :] = idx
    oh = (iota == idx[None, :]).astype(jnp.bfloat16)
    cnt = jnp.sum(oh, axis=1, dtype=jnp.float32)[None, :]
    counts_ref[:, :] = jnp.where(first_of_group, cnt, counts_ref[:, :] + cnt)
    cb_hi = cb.astype(jnp.bfloat16)
    cb_lo = (cb - cb_hi.astype(jnp.float32)).astype(jnp.bfloat16)
    cb_hl = jnp.concatenate([cb_hi, cb_lo], axis=1)
    zq2 = jax.lax.dot_general(
        cb_hl, oh, (((0,), (0,)), ((), ())),
        preferred_element_type=jnp.float32)
    dpg = xT.shape[0]
    zqT = zq2[:dpg, :] + zq2[dpg:, :]
    quant_ref[0] = zqT
    lacc_ref[0] += jnp.sum((zqT - xT) ** 2)

    @pl.when(last_of_group)
    def _fin_group():
        t = n_b * n_s * xT.shape[1]
        probs = counts_ref[0, :] * (1.0 / t)
        ent = -jnp.sum(probs * jnp.log(probs + 1e-10))
        lane4 = jax.lax.broadcasted_iota(jnp.int32, (GROUPS_K,), 0)
        prev = jnp.where(g == 0, jnp.zeros((GROUPS_K,), jnp.float32),
                         perps_ref[0, :])
        perps_ref[0, :] = jnp.where(lane4 == g, jnp.exp(ent), prev)

    @pl.when(jnp.logical_and(g == GROUPS_K - 1, last_of_group))
    def _fin_all():
        total = (1.0 + BETA_K) * lacc_ref[0] / n_elems
        loss_ref[:, :] = jnp.full((1, 1), total, jnp.float32)


def kernel(z, codebooks):
    z = z.astype(jnp.float32)
    B, C, H, W = z.shape
    HW = H * W
    G, N_E, DPG = codebooks.shape
    zr = z.reshape(B, C, HW)
    S = min(4096, HW)
    n_s = HW // S
    grid = (G, B, n_s)
    T = B * HW
    n_elems = T * DPG

    body = functools.partial(_vq_body, n_b=B, n_s=n_s, n_e=N_E,
                             n_elems=float(n_elems))
    quant, inds4, loss, perps = pl.pallas_call(
        body,
        grid=grid,
        in_specs=[
            pl.BlockSpec((1, DPG, S), lambda g, b, s: (b, g, s)),
            pl.BlockSpec((1, N_E, DPG), lambda g, b, s: (g, 0, 0)),
        ],
        out_specs=[
            pl.BlockSpec((1, DPG, S), lambda g, b, s: (b, g, s)),
            pl.BlockSpec((1, 1, 1, S), lambda g, b, s: (g, b, 0, s)),
            pl.BlockSpec((1, 1), lambda g, b, s: (0, 0)),
            pl.BlockSpec((1, GROUPS_K), lambda g, b, s: (0, 0)),
        ],
        out_shape=[
            jax.ShapeDtypeStruct((B, C, HW), jnp.float32),
            jax.ShapeDtypeStruct((G, B, 1, HW), jnp.int32),
            jax.ShapeDtypeStruct((1, 1), jnp.float32),
            jax.ShapeDtypeStruct((1, GROUPS_K), jnp.float32),
        ],
        scratch_shapes=[
            pltpu.VMEM((1, N_E), jnp.float32),
            pltpu.SMEM((1,), jnp.float32),
        ],
    )(zr, codebooks)

    quantized = quant.reshape(B, C, H, W)
    inds = inds4.reshape(G, B, HW)
    return (quantized, loss[0, 0], perps[0], inds)
